# Initial kernel scaffold; baseline (speedup 1.0000x reference)
import jax
import jax.numpy as jnp
from jax.experimental import pallas as pl


def kernel(x, edge_index, W1, as1, ad1, b1, W2, as2, ad2, b2, Wmu, asmu, admu, bmu, Wls, asls, adls, bls):
    raise NotImplementedError("write your pallas kernel here")


# speedup vs baseline: 1.0330x; 1.0330x over previous
import functools

import jax
import jax.numpy as jnp
from jax.experimental import pallas as pl
from jax.experimental.pallas import tpu as pltpu


def _matmul_kernel(x_ref, w_ref, o_ref):
    o_ref[...] = jnp.dot(x_ref[...], w_ref[...],
                         preferred_element_type=jnp.float32)


def _pallas_matmul(x, w):
    n, k = x.shape
    k2, m = w.shape
    bn = 1000
    return pl.pallas_call(
        _matmul_kernel,
        grid=(n // bn,),
        in_specs=[
            pl.BlockSpec((bn, k), lambda i: (i, 0)),
            pl.BlockSpec((k, m), lambda i: (0, 0)),
        ],
        out_specs=pl.BlockSpec((bn, m), lambda i: (i, 0)),
        out_shape=jax.ShapeDtypeStruct((n, m), jnp.float32),
    )(x, w)


def _gat(x, edge_index, W, a_src, a_dst, bias, heads, out_ch, concat):
    N = x.shape[0]
    loops = jnp.arange(N, dtype=edge_index.dtype)
    src = jnp.concatenate([edge_index[0], loops])
    dst = jnp.concatenate([edge_index[1], loops])
    h = _pallas_matmul(x, W).reshape(N, heads, out_ch)
    alpha_s = (h * a_src[None, :, :]).sum(-1)
    alpha_d = (h * a_dst[None, :, :]).sum(-1)
    e = alpha_s[src] + alpha_d[dst]
    e = jax.nn.leaky_relu(e, 0.2)
    emax = jax.ops.segment_max(e, dst, num_segments=N)
    e = jnp.exp(e - emax[dst])
    denom = jax.ops.segment_sum(e, dst, num_segments=N)
    alpha = e / (denom[dst] + 1e-16)
    msg = h[src] * alpha[:, :, None]
    out = jax.ops.segment_sum(msg, dst, num_segments=N)
    if concat:
        out = out.reshape(N, heads * out_ch)
    else:
        out = out.mean(axis=1)
    return out + bias


def kernel(x, edge_index, W1, as1, ad1, b1, W2, as2, ad2, b2,
           Wmu, asmu, admu, bmu, Wls, asls, adls, bls):
    h = jax.nn.elu(_gat(x, edge_index, W1, as1, ad1, b1, 8, 64, True))
    h = jax.nn.elu(_gat(h, edge_index, W2, as2, ad2, b2, 1, 64, False))
    mu = _gat(h, edge_index, Wmu, asmu, admu, bmu, 1, 32, False)
    logstd = _gat(h, edge_index, Wls, asls, adls, bls, 1, 32, False)
    return (mu, logstd)


# speedup vs baseline: 9.4142x; 9.1132x over previous
import functools

import jax
import jax.numpy as jnp
from jax import lax
from jax.experimental import pallas as pl
from jax.experimental.pallas import tpu as pltpu
from jax.experimental.pallas import tpu_sc as plsc

N = 10000
E = 320000
NC, NS, NL = 2, 16, 16
NW = NC * NS
EW = E // NW
NBLK = EW // NL
SUB_ROWS = 632
N_PAD = SUB_ROWS * NS


def _shuffle(v, idxvec):
    dn = lax.GatherDimensionNumbers(
        offset_dims=(), collapsed_slice_dims=(0,), start_index_map=(0,))
    return lax.gather(v, idxvec.reshape(NL, 1), dn, (1,),
                      mode=lax.GatherScatterMode.PROMISE_IN_BOUNDS)


def _bcast_lane(v, r):
    return _shuffle(v, jnp.full((NL,), r, jnp.int32))


@functools.cache
def _make_edge_pass(hp, d, interpret=False):
    dh = d // hp
    mesh = plsc.VectorSubcoreMesh(core_axis_name="c", subcore_axis_name="s",
                                  num_cores=NC, num_subcores=NS)

    @functools.partial(
        pl.kernel,
        out_type=[
            jax.ShapeDtypeStruct((NC, N_PAD, d), jnp.float32),
            jax.ShapeDtypeStruct((NC, N_PAD, 16), jnp.float32),
        ],
        mesh=mesh,
        scratch_types=[
            pltpu.VMEM_SHARED((N_PAD, d), jnp.float32),
            pltpu.VMEM_SHARED((N_PAD, 16), jnp.float32),
            pltpu.VMEM((EW,), jnp.int32),
            pltpu.VMEM((EW,), jnp.int32),
            pltpu.VMEM((NL, d), jnp.float32),
            pltpu.VMEM((NL, d), jnp.float32),
            pltpu.VMEM((NL, 16), jnp.float32),
            pltpu.VMEM((NL, 16), jnp.float32),
            pltpu.VMEM((NL, 16), jnp.float32),
            pltpu.SemaphoreType.DMA,
            pltpu.SemaphoreType.DMA,
            pltpu.SemaphoreType.DMA,
        ],
        compiler_params=pltpu.CompilerParams(needs_layout_passes=False,
                                             use_tc_tiling_on_sc=False),
        interpret=interpret,
    )
    def edge_pass(src_h, dst_h, sst_h, ddt_h, v_h, acc_h, den_h,
                  acc_s, den_s, src_v, dst_v,
                  vbuf, tbuf, sbuf, dbuf, denrows, sem_v, sem_s, sem_d):
        cid = lax.axis_index("c")
        sid = lax.axis_index("s")
        wid = sid * NC + cid

        zero = jnp.zeros((NL,), jnp.float32)
        for r in range(NL):
            for j in range(d // NL):
                tbuf[r, pl.ds(j * NL, NL)] = zero
            denrows[r, :] = zero
        row0 = sid * SUB_ROWS

        def zero_body(k, _):
            pltpu.sync_copy(tbuf.at[pl.ds(0, 8)],
                            acc_s.at[pl.ds(row0 + k * 8, 8)])
            pltpu.sync_copy(denrows.at[pl.ds(0, 8)],
                            den_s.at[pl.ds(row0 + k * 8, 8)])
            return _
        lax.fori_loop(0, SUB_ROWS // 8, zero_body, None)

        pltpu.sync_copy(src_h.at[pl.ds(wid * EW, EW)], src_v)
        pltpu.sync_copy(dst_h.at[pl.ds(wid * EW, EW)], dst_v)

        plsc.subcore_barrier()

        lanes = lax.iota(jnp.int32, NL)
        lt_hp = lanes < hp
        rot = hp + (lanes % hp if hp > 1 else lanes * 0)

        def body(b, _):
            sidx = src_v[pl.ds(b * NL, NL)]
            didx = dst_v[pl.ds(b * NL, NL)]
            pltpu.async_copy(v_h.at[sidx], vbuf, sem_v).wait()
            pltpu.async_copy(sst_h.at[sidx], sbuf, sem_s).wait()
            pltpu.async_copy(ddt_h.at[didx], dbuf, sem_d).wait()
            for r in range(NL):
                srow = sbuf[r, :]
                drow = dbuf[r, :]
                e = srow + drow
                e = jnp.where(e >= 0, e, 0.2 * e)
                w = jnp.exp(e - _shuffle(drow, rot))
                w = jnp.where(lt_hp, w, 0.0)
                denrows[r, :] = w
                for h in range(hp):
                    wb = _bcast_lane(w, h)
                    for j in range(dh // NL):
                        sl = pl.ds(h * dh + j * NL, NL)
                        tbuf[r, sl] = wb * vbuf[r, sl]
            pltpu.sync_copy(tbuf, acc_s.at[didx], add=True)
            pltpu.sync_copy(denrows, den_s.at[didx], add=True)
            return _

        lax.fori_loop(0, NBLK, body, None)

        plsc.subcore_barrier()
        pltpu.sync_copy(acc_s.at[pl.ds(row0, SUB_ROWS)],
                        acc_h.at[cid, pl.ds(row0, SUB_ROWS)])
        pltpu.sync_copy(den_s.at[pl.ds(row0, SUB_ROWS)],
                        den_h.at[cid, pl.ds(row0, SUB_ROWS)])

    return edge_pass


def _lrelu(x):
    return jnp.where(x >= 0, x, 0.2 * x)


def _attend(src, dst, ssrc, sdst, v, hp):
    d = v.shape[1]
    smax = jnp.max(ssrc, axis=0)
    c = _lrelu(smax[None, :] + sdst)
    sst = jnp.zeros((N, 16), jnp.float32).at[:, :hp].set(ssrc)
    ddt = (jnp.zeros((N, 16), jnp.float32).at[:, :hp].set(sdst)
           .at[:, hp:2 * hp].set(c))
    acc, den = _make_edge_pass(hp, d)(src, dst, sst, ddt, v)
    w_self = jnp.exp(_lrelu(ssrc + sdst) - c)
    num = acc[0, :N] + acc[1, :N]
    num = num + jnp.repeat(w_self, d // hp, axis=1) * v
    dent = den[0, :N, :hp] + den[1, :N, :hp] + w_self + 1e-16
    return num / jnp.repeat(dent, d // hp, axis=1)


def kernel(x, edge_index, W1, as1, ad1, b1, W2, as2, ad2, b2,
           Wmu, asmu, admu, bmu, Wls, asls, adls, bls):
    src, dst = edge_index[0], edge_index[1]

    h1 = (x @ W1).reshape(N, 8, 64)
    s1s = (h1 * as1[None]).sum(-1)
    s1d = (h1 * ad1[None]).sum(-1)
    outs = []
    for p in range(4):
        v = h1[:, 2 * p:2 * p + 2, :].reshape(N, 128)
        agg = _attend(src, dst, s1s[:, 2 * p:2 * p + 2],
                      s1d[:, 2 * p:2 * p + 2], v, 2)
        outs.append(agg)
    h = jax.nn.elu(jnp.concatenate(outs, axis=1) + b1)

    h2p = h @ W2
    s2s = (h2p * as2[0][None]).sum(-1, keepdims=True)
    s2d = (h2p * ad2[0][None]).sum(-1, keepdims=True)
    h = jax.nn.elu(_attend(src, dst, s2s, s2d, h2p, 1) + b2)

    pmu = h @ Wmu
    pls = h @ Wls
    sms = (pmu * asmu[0][None]).sum(-1, keepdims=True)
    smd = (pmu * admu[0][None]).sum(-1, keepdims=True)
    sls = (pls * asls[0][None]).sum(-1, keepdims=True)
    sld = (pls * adls[0][None]).sum(-1, keepdims=True)
    vml = jnp.concatenate([pmu, pls], axis=1)
    ssrc = jnp.concatenate([sms, sls], axis=1)
    sdst = jnp.concatenate([smd, sld], axis=1)
    agg = _attend(src, dst, ssrc, sdst, vml, 2)
    mu = agg[:, :32] + bmu
    logstd = agg[:, 32:] + bls
    return (mu, logstd)


# speedup vs baseline: 31.3423x; 3.3293x over previous
import functools

import jax
import jax.numpy as jnp
from jax import lax
from jax.experimental import pallas as pl
from jax.experimental.pallas import tpu as pltpu
from jax.experimental.pallas import tpu_sc as plsc

N = 10000
E = 320000
NC, NS, NL = 2, 16, 16
NW = NC * NS
EW = E // NW
NBLK = EW // NL
NBLKP = 628
EWP = NBLKP * NL
SUB_ROWS = 632
N_PAD = SUB_ROWS * NS


def _shuffle(v, idxvec):
    dn = lax.GatherDimensionNumbers(
        offset_dims=(), collapsed_slice_dims=(0,), start_index_map=(0,))
    return lax.gather(v, idxvec.reshape(NL, 1), dn, (1,),
                      mode=lax.GatherScatterMode.PROMISE_IN_BOUNDS)


def _bcast_lane(v, r):
    return _shuffle(v, jnp.full((NL,), r, jnp.int32))


@functools.cache
def _make_edge_pass(hp, d, interpret=False):
    dh = d // hp
    mesh = plsc.VectorSubcoreMesh(core_axis_name="c", subcore_axis_name="s",
                                  num_cores=NC, num_subcores=NS)

    @functools.partial(
        pl.kernel,
        out_type=[
            jax.ShapeDtypeStruct((NC, N_PAD, d), jnp.float32),
            jax.ShapeDtypeStruct((NC, N_PAD, 16), jnp.float32),
        ],
        mesh=mesh,
        scratch_types=[
            pltpu.VMEM_SHARED((N_PAD, d), jnp.float32),
            pltpu.VMEM_SHARED((N_PAD, 16), jnp.float32),
            pltpu.VMEM((EWP,), jnp.int32),
            pltpu.VMEM((EWP,), jnp.int32),
            [pltpu.VMEM((NL, d), jnp.float32)] * 2,
            [pltpu.VMEM((NL, d), jnp.float32)] * 2,
            [pltpu.VMEM((NL, 16), jnp.float32)] * 2,
            [pltpu.VMEM((NL, 16), jnp.float32)] * 2,
            [pltpu.VMEM((NL, 16), jnp.float32)] * 2,
            [pltpu.SemaphoreType.DMA] * 2,
            [pltpu.SemaphoreType.DMA] * 2,
        ],
        compiler_params=pltpu.CompilerParams(needs_layout_passes=False,
                                             use_tc_tiling_on_sc=False),
        interpret=interpret,
    )
    def edge_pass(src_h, dst_h, sst_h, ddt_h, v_h, acc_h, den_h,
                  acc_s, den_s, src_v, dst_v,
                  vbuf, tbuf, sbuf, dbuf, denrows, gsem, ssem):
        cid = lax.axis_index("c")
        sid = lax.axis_index("s")
        wid = sid * NC + cid

        zero = jnp.zeros((NL,), jnp.float32)
        for q in range(2):
            for r in range(NL):
                for j in range(d // NL):
                    tbuf[q][r, pl.ds(j * NL, NL)] = zero
                denrows[q][r, :] = zero
        row0 = sid * SUB_ROWS

        def zero_body(k, _):
            pltpu.sync_copy(tbuf[0].at[pl.ds(0, 8)],
                            acc_s.at[pl.ds(row0 + k * 8, 8)])
            pltpu.sync_copy(denrows[0].at[pl.ds(0, 8)],
                            den_s.at[pl.ds(row0 + k * 8, 8)])
            return _
        lax.fori_loop(0, SUB_ROWS // 8, zero_body, None)

        pltpu.sync_copy(src_h.at[pl.ds(wid * EWP, EWP)], src_v)
        pltpu.sync_copy(dst_h.at[pl.ds(wid * EWP, EWP)], dst_v)

        plsc.subcore_barrier()

        lanes = lax.iota(jnp.int32, NL)
        lt_hp = lanes < hp
        rot = hp + (lanes % hp if hp > 1 else lanes * 0)

        def start_gathers(b, q):
            sidx = src_v[pl.ds(b * NL, NL)]
            didx = dst_v[pl.ds(b * NL, NL)]
            pltpu.make_async_copy(v_h.at[sidx], vbuf[q], gsem[q]).start()
            pltpu.make_async_copy(sst_h.at[sidx], sbuf[q], gsem[q]).start()
            pltpu.make_async_copy(ddt_h.at[didx], dbuf[q], gsem[q]).start()

        def wait_gathers(q):
            sidx = src_v[pl.ds(0, NL)]
            pltpu.make_async_copy(v_h.at[sidx], vbuf[q], gsem[q]).wait()
            pltpu.make_async_copy(sst_h.at[sidx], sbuf[q], gsem[q]).wait()
            pltpu.make_async_copy(ddt_h.at[sidx], dbuf[q], gsem[q]).wait()

        def start_scatters(b, q):
            didx = dst_v[pl.ds(b * NL, NL)]
            pltpu.make_async_copy(tbuf[q], acc_s.at[didx],
                                  ssem[q]).start(add=True)
            pltpu.make_async_copy(denrows[q], den_s.at[didx],
                                  ssem[q]).start(add=True)

        def wait_scatters(q):
            didx = dst_v[pl.ds(0, NL)]
            pltpu.make_async_copy(tbuf[q], acc_s.at[didx], ssem[q]).wait()
            pltpu.make_async_copy(denrows[q], den_s.at[didx], ssem[q]).wait()

        def compute(q):
            for r in range(NL):
                srow = sbuf[q][r, :]
                drow = dbuf[q][r, :]
                e = srow + drow
                e = jnp.where(e >= 0, e, 0.2 * e)
                w = jnp.exp(e - _shuffle(drow, rot))
                w = jnp.where(lt_hp, w, 0.0)
                denrows[q][r, :] = w
                for h in range(hp):
                    wb = _bcast_lane(w, h)
                    for j in range(dh // NL):
                        sl = pl.ds(h * dh + j * NL, NL)
                        tbuf[q][r, sl] = wb * vbuf[q][r, sl]

        start_gathers(0, 0)
        start_scatters(0, 0)
        start_scatters(0, 1)

        def pair_body(i, _):
            b0 = i * 2
            start_gathers(b0 + 1, 1)
            wait_gathers(0)
            wait_scatters(0)
            compute(0)
            start_scatters(b0, 0)
            start_gathers(b0 + 2, 0)
            wait_gathers(1)
            wait_scatters(1)
            compute(1)
            start_scatters(b0 + 1, 1)
            return _

        lax.fori_loop(0, 313, pair_body, None)

        wait_gathers(0)
        wait_scatters(0)
        wait_scatters(1)

        plsc.subcore_barrier()
        pltpu.sync_copy(acc_s.at[pl.ds(row0, SUB_ROWS)],
                        acc_h.at[cid, pl.ds(row0, SUB_ROWS)])
        pltpu.sync_copy(den_s.at[pl.ds(row0, SUB_ROWS)],
                        den_h.at[cid, pl.ds(row0, SUB_ROWS)])

    return edge_pass


def _lrelu(x):
    return jnp.where(x >= 0, x, 0.2 * x)


def _attend(src, dst, ssrc, sdst, v, hp):
    d = v.shape[1]
    smax = jnp.max(ssrc, axis=0)
    c = _lrelu(smax[None, :] + sdst)
    sst = jnp.zeros((N_PAD, 16), jnp.float32).at[:N, :hp].set(ssrc)
    ddt = (jnp.zeros((N_PAD, 16), jnp.float32).at[:N, :hp].set(sdst)
           .at[:N, hp:2 * hp].set(c))
    acc, den = _make_edge_pass(hp, d)(src, dst, sst, ddt, v)
    w_self = jnp.exp(_lrelu(ssrc + sdst) - c)
    num = acc[0, :N] + acc[1, :N]
    num = num + jnp.repeat(w_self, d // hp, axis=1) * v
    dent = den[0, :N, :hp] + den[1, :N, :hp] + w_self + 1e-16
    return num / jnp.repeat(dent, d // hp, axis=1)


def kernel(x, edge_index, W1, as1, ad1, b1, W2, as2, ad2, b2,
           Wmu, asmu, admu, bmu, Wls, asls, adls, bls):
    pad_s = jnp.zeros((NW, EWP - EW), jnp.int32)
    pad_d = jnp.full((NW, EWP - EW), N, jnp.int32)
    src = jnp.concatenate([edge_index[0].reshape(NW, EW), pad_s],
                          axis=1).reshape(-1)
    dst = jnp.concatenate([edge_index[1].reshape(NW, EW), pad_d],
                          axis=1).reshape(-1)

    h1 = (x @ W1).reshape(N, 8, 64)
    s1s = (h1 * as1[None]).sum(-1)
    s1d = (h1 * ad1[None]).sum(-1)
    outs = []
    for p in range(4):
        v = h1[:, 2 * p:2 * p + 2, :].reshape(N, 128)
        agg = _attend(src, dst, s1s[:, 2 * p:2 * p + 2],
                      s1d[:, 2 * p:2 * p + 2], v, 2)
        outs.append(agg)
    h = jax.nn.elu(jnp.concatenate(outs, axis=1) + b1)

    h2p = h @ W2
    s2s = (h2p * as2[0][None]).sum(-1, keepdims=True)
    s2d = (h2p * ad2[0][None]).sum(-1, keepdims=True)
    h = jax.nn.elu(_attend(src, dst, s2s, s2d, h2p, 1) + b2)

    pmu = h @ Wmu
    pls = h @ Wls
    sms = (pmu * asmu[0][None]).sum(-1, keepdims=True)
    smd = (pmu * admu[0][None]).sum(-1, keepdims=True)
    sls = (pls * asls[0][None]).sum(-1, keepdims=True)
    sld = (pls * adls[0][None]).sum(-1, keepdims=True)
    vml = jnp.concatenate([pmu, pls], axis=1)
    ssrc = jnp.concatenate([sms, sls], axis=1)
    sdst = jnp.concatenate([smd, sld], axis=1)
    agg = _attend(src, dst, ssrc, sdst, vml, 2)
    mu = agg[:, :32] + bmu
    logstd = agg[:, 32:] + bls
    return (mu, logstd)
